# baseline (speedup 1.0000x reference)
import jax
import jax.numpy as jnp
from jax import lax
from jax.experimental import pallas as pl
from jax.experimental.pallas import tpu as pltpu
from jax.experimental.pallas import tpu_sc as plsc

B = 16384
D = 16
L = 16
NC = 2
NS = 16
NW = NC * NS
BPW = B // NW
G = 2
NSLOT = 8
NFLY = 7
NG = BPW // G
W = 128
WSH = 7
WM = W - 1

_mesh = plsc.VectorSubcoreMesh(core_axis_name="c", subcore_axis_name="s")

_SCRATCH = [
    pltpu.VMEM((BPW + L,), jnp.int32),
    pltpu.VMEM((BPW + L,), jnp.int32),
    pltpu.VMEM((NSLOT, G, D, W), jnp.float32),
    pltpu.VMEM((NSLOT, G, D, W), jnp.float32),
    pltpu.VMEM((BPW,), jnp.float32),
] + [pltpu.SemaphoreType.DMA] * (2 * NSLOT)


def _sc_dot_body(user_hbm, item_hbm, vtu_hbm, vti_hbm, out_hbm,
                 uidx_v, iidx_v, ubuf, ibuf, out_v, *sems):
    usems = sems[:NSLOT]
    isems = sems[NSLOT:]
    wid = lax.axis_index("s") * NC + lax.axis_index("c")
    base = wid * BPW

    pltpu.sync_copy(user_hbm.at[pl.ds(base, BPW)], uidx_v.at[pl.ds(0, BPW)])
    pltpu.sync_copy(item_hbm.at[pl.ds(base, BPW)], iidx_v.at[pl.ds(0, BPW)])

    lane = lax.iota(jnp.int32, L)

    def issue_group(g, slot):
        g = jnp.minimum(g, NG - 1)
        ug = uidx_v[pl.ds(g * G, L)]
        ig = iidx_v[pl.ds(g * G, L)]
        uw = (ug >> WSH) << WSH
        iw = (ig >> WSH) << WSH
        for k in range(G):
            wu = pl.multiple_of(uw[k], W)
            wi = pl.multiple_of(iw[k], W)
            for h in range(2):
                hs = pl.ds(h * 8, 8)
                pltpu.async_copy(vtu_hbm.at[hs, pl.ds(wu, W)],
                                 ubuf.at[slot, k, hs], usems[slot])
                pltpu.async_copy(vti_hbm.at[hs, pl.ds(wi, W)],
                                 ibuf.at[slot, k, hs], isems[slot])

    def wait_group(slot):
        for k in range(G):
            pltpu.make_async_copy(vtu_hbm.at[:, pl.ds(0, W)],
                                  ubuf.at[slot, k], usems[slot]).wait()
            pltpu.make_async_copy(vti_hbm.at[:, pl.ds(0, W)],
                                  ibuf.at[slot, k], isems[slot]).wait()

    for p in range(NFLY):
        issue_group(p, p)

    def blk_body(m, carry):
        res = jnp.zeros((L,), jnp.float32)
        u16 = uidx_v[pl.ds(m * L, L)]
        i16 = iidx_v[pl.ds(m * L, L)]
        ulo = u16 & WM
        ilo = i16 & WM
        for b in range(NSLOT):
            g = m * NSLOT + b
            wait_group(b)
            issue_group(g + NFLY, (b + NFLY) % NSLOT)
            for k in range(G):
                t = b * G + k
                uvec = jnp.broadcast_to(ulo[t], (L,))
                ivec = jnp.broadcast_to(ilo[t], (L,))
                urow = plsc.load_gather(ubuf.at[b, k], [lane, uvec])
                irow = plsc.load_gather(ibuf.at[b, k], [lane, ivec])
                s = jnp.sum(urow * irow)
                res = jnp.where(lane == t, s, res)
        out_v[pl.ds(m * L, L)] = 1.0 / (1.0 + jnp.exp(-res))
        return carry

    lax.fori_loop(0, NG // NSLOT, blk_body, 0)

    for p in range(NFLY):
        wait_group(p)

    pltpu.sync_copy(out_v, out_hbm.at[pl.ds(base, BPW)])


_sc_dot = pl.kernel(
    _sc_dot_body,
    mesh=_mesh,
    compiler_params=pltpu.CompilerParams(needs_layout_passes=False),
    out_type=jax.ShapeDtypeStruct((B,), jnp.float32),
    scratch_types=_SCRATCH,
)


def kernel(user, item, user_table, item_table):
    vtu = user_table.T
    vti = item_table.T
    return _sc_dot(user.astype(jnp.int32), item.astype(jnp.int32), vtu, vti)


# speedup vs baseline: 1.0336x; 1.0336x over previous
import jax
import jax.numpy as jnp
from jax import lax
from jax.experimental import pallas as pl
from jax.experimental.pallas import tpu as pltpu
from jax.experimental.pallas import tpu_sc as plsc

B = 16384
D = 16
L = 16
NC = 2
NS = 16
NW = NC * NS
BPW = B // NW
G = 4
NSLOT = 4
NG = BPW // G
W = 128
WSH = 7
WM = W - 1

_mesh = plsc.VectorSubcoreMesh(core_axis_name="c", subcore_axis_name="s")

_SCRATCH = [
    pltpu.VMEM((BPW + L,), jnp.int32),
    pltpu.VMEM((BPW + L,), jnp.int32),
    pltpu.VMEM((NSLOT, G, D, W), jnp.float32),
    pltpu.VMEM((NSLOT, G, D, W), jnp.float32),
    pltpu.VMEM((BPW,), jnp.float32),
] + [pltpu.SemaphoreType.DMA] * (2 * NSLOT)


def _sc_dot_body(user_hbm, item_hbm, vtu_hbm, vti_hbm, out_hbm,
                 uidx_v, iidx_v, ubuf, ibuf, out_v, *sems):
    usems = sems[:NSLOT]
    isems = sems[NSLOT:]
    wid = lax.axis_index("s") * NC + lax.axis_index("c")
    base = wid * BPW

    pltpu.sync_copy(user_hbm.at[pl.ds(base, BPW)], uidx_v.at[pl.ds(0, BPW)])
    pltpu.sync_copy(item_hbm.at[pl.ds(base, BPW)], iidx_v.at[pl.ds(0, BPW)])

    lane = lax.iota(jnp.int32, L)

    def issue_group(g, slot):
        g = jnp.minimum(g, NG - 1)
        ug = uidx_v[pl.ds(g * G, L)]
        ig = iidx_v[pl.ds(g * G, L)]
        uw = (ug >> WSH) << WSH
        iw = (ig >> WSH) << WSH
        for k in range(G):
            wu = pl.multiple_of(uw[k], W)
            wi = pl.multiple_of(iw[k], W)
            for h in range(2):
                hs = pl.ds(h * 8, 8)
                pltpu.async_copy(vtu_hbm.at[hs, pl.ds(wu, W)],
                                 ubuf.at[slot, k, hs], usems[slot])
                pltpu.async_copy(vti_hbm.at[hs, pl.ds(wi, W)],
                                 ibuf.at[slot, k, hs], isems[slot])

    def wait_group(slot):
        for k in range(G):
            pltpu.make_async_copy(vtu_hbm.at[:, pl.ds(0, W)],
                                  ubuf.at[slot, k], usems[slot]).wait()
            pltpu.make_async_copy(vti_hbm.at[:, pl.ds(0, W)],
                                  ibuf.at[slot, k], isems[slot]).wait()

    issue_group(0, 0)
    issue_group(1, 1)
    issue_group(2, 2)

    def blk_body(m, carry):
        res = jnp.zeros((L,), jnp.float32)
        u16 = uidx_v[pl.ds(m * L, L)]
        i16 = iidx_v[pl.ds(m * L, L)]
        ulo = u16 & WM
        ilo = i16 & WM
        for b in range(NSLOT):
            g = m * NSLOT + b
            wait_group(b)
            issue_group(g + 3, (b + 3) % NSLOT)
            for k in range(G):
                t = b * G + k
                uvec = jnp.broadcast_to(ulo[t], (L,))
                ivec = jnp.broadcast_to(ilo[t], (L,))
                urow = plsc.load_gather(ubuf.at[b, k], [lane, uvec])
                irow = plsc.load_gather(ibuf.at[b, k], [lane, ivec])
                s = jnp.sum(urow * irow)
                res = jnp.where(lane == t, s, res)
        out_v[pl.ds(m * L, L)] = 1.0 / (1.0 + jnp.exp(-res))
        return carry

    lax.fori_loop(0, NG // NSLOT, blk_body, 0)

    wait_group(0)
    wait_group(1)
    wait_group(2)

    pltpu.sync_copy(out_v, out_hbm.at[pl.ds(base, BPW)])


_sc_dot = pl.kernel(
    _sc_dot_body,
    mesh=_mesh,
    compiler_params=pltpu.CompilerParams(needs_layout_passes=False),
    out_type=jax.ShapeDtypeStruct((B,), jnp.float32),
    scratch_types=_SCRATCH,
)


def kernel(user, item, user_table, item_table):
    vtu = user_table.T
    vti = item_table.T
    return _sc_dot(user.astype(jnp.int32), item.astype(jnp.int32), vtu, vti)
